# baseline (speedup 1.0000x reference)
import functools

import jax
import jax.numpy as jnp
from jax import lax
from jax.experimental import pallas as pl
from jax.experimental.pallas import tpu as pltpu
from jax.experimental.pallas import tpu_sc as plsc

NUM_HEADS = 16
NUM_BUCKETS = 32
SEQ = 2048
WS = 3968
DPAD = 4224
NSHIFT = 16


def _table_body(delta_ref, embt_ref, out_ref, vflat_ref):
    m = pl.program_id(0)

    @pl.when(m == 0)
    def _():
        delta = delta_ref[0, 0]
        d = lax.broadcasted_iota(jnp.int32, (1, DPAD), 1)
        rel = d - (SEQ - 1) + delta
        n = -rel
        side = jnp.where(n < 0, 16, 0).astype(jnp.int32)
        na = jnp.abs(n)
        large = jnp.full(na.shape, 8, jnp.int32)
        for j in range(1, 16):
            large = large + (na >= (1 << (j + 3))).astype(jnp.int32)
        bucket = side + jnp.where(na < 8, na, large)
        bucket = jnp.minimum(bucket, NUM_BUCKETS - 1)
        rows = lax.broadcasted_iota(jnp.int32, (NUM_BUCKETS, DPAD), 0)
        oh = (jnp.broadcast_to(bucket, (NUM_BUCKETS, DPAD)) == rows
              ).astype(jnp.float32)
        vflat_ref[...] = lax.dot_general(
            embt_ref[...], oh, (((1,), (0,)), ((), ())),
            precision=lax.Precision.HIGHEST,
            preferred_element_type=jnp.float32)

    vf = vflat_ref[...]
    cur = pltpu.roll(vf, DPAD - (8 * m + 7), 1)
    for r in range(8):
        out_ref[:, 0, r, :] = cur[:, :WS]
        if r < 7:
            cur = pltpu.roll(cur, 1, 1)


def _build_table(embt, delta):
    return pl.pallas_call(
        _table_body,
        grid=(NSHIFT,),
        out_shape=jax.ShapeDtypeStruct((NUM_HEADS, NSHIFT, 8, WS), jnp.float32),
        in_specs=[
            pl.BlockSpec(memory_space=pltpu.SMEM),
            pl.BlockSpec(memory_space=pltpu.VMEM),
        ],
        out_specs=pl.BlockSpec((NUM_HEADS, 1, 8, WS), lambda m: (0, m, 0, 0)),
        scratch_shapes=[pltpu.VMEM((NUM_HEADS, DPAD), jnp.float32)],
    )(delta, embt)


@functools.cache
def _make_sc_expand():
    mesh = plsc.VectorSubcoreMesh(core_axis_name="c", subcore_axis_name="s")

    @functools.partial(
        pl.kernel,
        mesh=mesh,
        out_type=jax.ShapeDtypeStruct((1, NUM_HEADS, SEQ, SEQ), jnp.float32),
        scratch_types=[
            pltpu.VMEM((8, WS), jnp.float32),
            pltpu.VMEM((8, WS), jnp.float32),
            pltpu.VMEM((8, WS), jnp.float32),
            pltpu.VMEM((8, WS), jnp.float32),
            pltpu.SemaphoreType.DMA,
            pltpu.SemaphoreType.DMA,
        ],
    )
    def _sc_expand(vshm_hbm, out_hbm, b0, b1, b2, b3, sem, ldsem):
        h = lax.axis_index("s")
        z = lax.axis_index("c")
        bufs = [b0, b1, b2, b3]
        depth = 16

        def class_m(pas, k):
            return 2 * (4 * pas + k) + 1 - z

        loads = [pltpu.async_copy(vshm_hbm.at[h, class_m(0, k)], bufs[k], ldsem)
                 for k in range(4)]
        for ld in loads:
            ld.wait()

        pending = []
        for pas in range(2):
            for k in range(4):
                mk = class_m(pas, k)
                a0 = (255 - mk) & 15
                for n in range(16):
                    a = a0 + 16 * n
                    p0 = (2040 - 8 * a) >> 7
                    src = bufs[k].at[:, pl.ds(
                        pl.multiple_of(128 * p0, 128), SEQ)]
                    dst = out_hbm.at[0, h, pl.ds(pl.multiple_of(8 * a, 8), 8), :]
                    pending.append(pltpu.async_copy(src, dst, sem))
                    if len(pending) > depth:
                        pending.pop(0).wait()
            if pas == 0:
                loads = []
                for k in range(4):
                    while len(pending) > 16 * (3 - k):
                        pending.pop(0).wait()
                    loads.append(pltpu.async_copy(
                        vshm_hbm.at[h, class_m(1, k)], bufs[k], ldsem))
                for ld in loads:
                    ld.wait()
        for cp in pending:
            cp.wait()

    return _sc_expand


def kernel(seq_len_q, seq_len_k, embedding):
    delta = (jnp.asarray(seq_len_k, jnp.int32)
             - jnp.asarray(seq_len_q, jnp.int32)).reshape(1, 1)
    embt = jnp.transpose(embedding).astype(jnp.float32)
    vshm = _build_table(embt, delta)
    return _make_sc_expand()(vshm)


# speedup vs baseline: 1.2220x; 1.2220x over previous
import functools

import jax
import jax.numpy as jnp
from jax import lax
from jax.experimental import pallas as pl
from jax.experimental.pallas import tpu as pltpu
from jax.experimental.pallas import tpu_sc as plsc

NUM_HEADS = 16
NUM_BUCKETS = 32
SEQ = 2048
WS = 3968
VR = 4096
DPAD = 4224


def _table_body(delta_ref, embt_ref, out_ref):
    delta = delta_ref[0, 0]
    d = lax.broadcasted_iota(jnp.int32, (1, DPAD), 1)
    rel = d - (SEQ - 1) + delta
    n = -rel
    side = jnp.where(n < 0, 16, 0).astype(jnp.int32)
    na = jnp.abs(n)
    large = jnp.full(na.shape, 8, jnp.int32)
    for j in range(1, 16):
        large = large + (na >= (1 << (j + 3))).astype(jnp.int32)
    bucket = side + jnp.where(na < 8, na, large)
    bucket = jnp.minimum(bucket, NUM_BUCKETS - 1)
    rows = lax.broadcasted_iota(jnp.int32, (NUM_BUCKETS, DPAD), 0)
    oh = (jnp.broadcast_to(bucket, (NUM_BUCKETS, DPAD)) == rows
          ).astype(jnp.float32)
    vflat = lax.dot_general(
        embt_ref[...], oh, (((1,), (0,)), ((), ())),
        precision=lax.Precision.HIGHEST,
        preferred_element_type=jnp.float32)
    for h in range(NUM_HEADS):
        out_ref[pl.ds(h * VR, VR)] = vflat[h, :VR]


def _build_table(embt, delta):
    return pl.pallas_call(
        _table_body,
        out_shape=jax.ShapeDtypeStruct((NUM_HEADS * VR,), jnp.float32),
        in_specs=[
            pl.BlockSpec(memory_space=pltpu.SMEM),
            pl.BlockSpec(memory_space=pltpu.VMEM),
        ],
    )(delta, embt)


@functools.cache
def _make_sc_expand():
    mesh = plsc.VectorSubcoreMesh(core_axis_name="c", subcore_axis_name="s")

    @functools.partial(
        pl.kernel,
        mesh=mesh,
        compiler_params=pltpu.CompilerParams(needs_layout_passes=False),
        out_type=jax.ShapeDtypeStruct((1, NUM_HEADS, SEQ, SEQ), jnp.float32),
        scratch_types=[
            pltpu.VMEM((VR,), jnp.float32),
            pltpu.VMEM((8, WS), jnp.float32),
            pltpu.VMEM((8, WS), jnp.float32),
            pltpu.VMEM((8, WS), jnp.float32),
            pltpu.SemaphoreType.DMA,
            pltpu.SemaphoreType.DMA,
        ],
    )
    def _sc_expand(vrow_hbm, out_hbm, vrow, p0b, p1b, p2b, sem, ldsem):
        h = lax.axis_index("s")
        z = lax.axis_index("c")
        pltpu.async_copy(
            vrow_hbm.at[pl.ds(pl.multiple_of(h * VR, 8), VR)], vrow, ldsem
        ).wait()
        bufs = [p0b, p1b, p2b]
        iota16 = lax.iota(jnp.int32, 16)

        def construct(buf, mk):
            base = 8 * mk
            def body(i, carry):
                col = 16 * i + iota16
                for r in range(8):
                    vals = plsc.load_gather(vrow, [col + (base + 7 - r)])
                    plsc.store_scatter(
                        buf, [jnp.full((16,), r, jnp.int32), col], vals)
                return carry
            lax.fori_loop(0, WS // 16, body, 0)

        classes = [2 * j + 1 - z for j in range(8)]
        construct(bufs[0], classes[0])
        pending = []
        for j in range(8):
            mk = classes[j]
            a0 = (255 - mk) & 15
            buf = bufs[j % 3]
            handles = []
            for n in range(16):
                a = a0 + 16 * n
                pv = (2040 - 8 * a) >> 7
                src = buf.at[:, pl.ds(pl.multiple_of(128 * pv, 128), SEQ)]
                dst = out_hbm.at[0, h, pl.ds(pl.multiple_of(8 * a, 8), 8), :]
                handles.append(pltpu.async_copy(src, dst, sem))
            pending.append(handles)
            if j < 7:
                if j >= 2:
                    for cp in pending[j - 2]:
                        cp.wait()
                construct(bufs[(j + 1) % 3], classes[j + 1])
        for lst in pending[5:]:
            for cp in lst:
                cp.wait()

    return _sc_expand


def kernel(seq_len_q, seq_len_k, embedding):
    delta = (jnp.asarray(seq_len_k, jnp.int32)
             - jnp.asarray(seq_len_q, jnp.int32)).reshape(1, 1)
    embt = jnp.transpose(embedding).astype(jnp.float32)
    vrow = _build_table(embt, delta)
    return _make_sc_expand()(vrow)
